# baseline (speedup 1.0000x reference)
import functools

import jax
import jax.numpy as jnp
from jax import lax
from jax.experimental import pallas as pl
from jax.experimental.pallas import tpu as pltpu
from jax.experimental.pallas import tpu_sc as plsc

N_E = 8192
E_DIM = 320
S7 = 7
M = 1024
SROW = 6272
CBLK = 256
NSTEP = N_E // CBLK
BANK_BLK = N_E // NSTEP
BETA = 0.25


def _argmin_body(zt_ref, e_ref, idx_ref, val_ref, loss_ref,
                 zbf_ref, s1_ref, best_val_ref, best_idx_ref):
    j = pl.program_id(0)

    @pl.when(j == 0)
    def _init():
        zt = zt_ref[...]
        zbf_ref[...] = zt.astype(jnp.bfloat16)
        s1_ref[...] = jnp.sum(zt * zt, axis=0, keepdims=True)
        best_val_ref[...] = jnp.full((1, M), jnp.inf, jnp.float32)
        best_idx_ref[...] = jnp.zeros((1, M), jnp.int32)

    e_blk = e_ref[...]
    e_bf = e_blk.astype(jnp.bfloat16)
    mm = lax.dot_general(e_bf, zbf_ref[...], (((1,), (0,)), ((), ())),
                         preferred_element_type=jnp.float32)
    s2 = jnp.sum(e_blk * e_blk, axis=1, keepdims=True)
    d = (s2 + s1_ref[...]) - 2.0 * mm

    blk_min = jnp.min(d, axis=0, keepdims=True)
    row_ids = lax.broadcasted_iota(jnp.int32, (CBLK, M), 0)
    cand = jnp.where(d == blk_min, row_ids, jnp.int32(N_E))
    blk_arg = jnp.min(cand, axis=0, keepdims=True) + j * CBLK

    upd = blk_min < best_val_ref[...]
    best_idx_ref[...] = jnp.where(upd, blk_arg, best_idx_ref[...])
    best_val_ref[...] = jnp.where(upd, blk_min, best_val_ref[...])

    @pl.when(j == NSTEP - 1)
    def _fini():
        idx_ref[...] = best_idx_ref[...]
        val_ref[...] = best_val_ref[...]
        tot = jnp.sum(best_val_ref[...], keepdims=True)
        loss_ref[...] = (1.0 + BETA) * tot.reshape(1, 1) / (M * E_DIM)


def _tc_argmin(zt, emb):
    return pl.pallas_call(
        _argmin_body,
        grid=(NSTEP,),
        in_specs=[
            pl.BlockSpec((E_DIM, M), lambda j: (0, 0)),
            pl.BlockSpec((CBLK, E_DIM), lambda j: (j, 0)),
        ],
        out_specs=[
            pl.BlockSpec((1, M), lambda j: (0, 0)),
            pl.BlockSpec((1, M), lambda j: (0, 0)),
            pl.BlockSpec((1, 1), lambda j: (0, 0)),
        ],
        out_shape=[
            jax.ShapeDtypeStruct((1, M), jnp.int32),
            jax.ShapeDtypeStruct((1, M), jnp.float32),
            jax.ShapeDtypeStruct((1, 1), jnp.float32),
        ],
        scratch_shapes=[
            pltpu.VMEM((E_DIM, M), jnp.bfloat16),
            pltpu.VMEM((1, M), jnp.float32),
            pltpu.VMEM((1, M), jnp.float32),
            pltpu.VMEM((1, M), jnp.int32),
        ],
        compiler_params=pltpu.CompilerParams(
            dimension_semantics=("arbitrary",),
        ),
    )(zt, emb)


ZBLK = 1024
NZSTEP = N_E // ZBLK


def _dedup_zq_body(idx_col_ref, idx_row_ref, e_ref, src_ref, zq_ref, acc_ref):
    t = pl.program_id(0)

    @pl.when(t == 0)
    def _dedup():
        col_b = lax.broadcasted_iota(jnp.int32, (M, M), 1)
        eq = idx_col_ref[...] == idx_row_ref[...]
        cand = jnp.where(eq, col_b, jnp.int32(-1))
        src_ref[...] = jnp.max(cand, axis=1, keepdims=True)
        acc_ref[...] = jnp.zeros((M, E_DIM), jnp.float32)

    codes = lax.broadcasted_iota(jnp.int32, (1, ZBLK), 1) + t * ZBLK
    oh = (idx_col_ref[...] == codes).astype(jnp.bfloat16)
    e_bf = e_ref[...].astype(jnp.bfloat16)
    acc_ref[...] += lax.dot_general(oh, e_bf, (((1,), (0,)), ((), ())),
                                    preferred_element_type=jnp.float32)

    @pl.when(t == NZSTEP - 1)
    def _fini():
        zq_ref[...] = acc_ref[...]


def _tc_dedup_zq(idx_col, idx_row, emb):
    return pl.pallas_call(
        _dedup_zq_body,
        grid=(NZSTEP,),
        in_specs=[
            pl.BlockSpec((M, 1), lambda t: (0, 0)),
            pl.BlockSpec((1, M), lambda t: (0, 0)),
            pl.BlockSpec((ZBLK, E_DIM), lambda t: (t, 0)),
        ],
        out_specs=[
            pl.BlockSpec((M, 1), lambda t: (0, 0)),
            pl.BlockSpec((M, E_DIM), lambda t: (0, 0)),
        ],
        out_shape=[
            jax.ShapeDtypeStruct((M, 1), jnp.int32),
            jax.ShapeDtypeStruct((M, E_DIM), jnp.float32),
        ],
        scratch_shapes=[pltpu.VMEM((M, E_DIM), jnp.float32)],
        compiler_params=pltpu.CompilerParams(
            dimension_semantics=("arbitrary",),
        ),
    )(idx_col, idx_row, emb)


_NC = 2
_NS = 16
_NW = _NC * _NS
_RPW = M // _NW
_CH = 8
_NCH = _RPW // _CH


_SLAB = N_E // _NW
_L = 16


def _sc_body(dst_hbm, src_hbm, structures_hbm, bank_in_hbm, bank_out_hbm,
             dst_v, src_v, csem, ssem):
    wid = lax.axis_index("s") * _NC + lax.axis_index("c")
    lo = wid * _SLAB
    pltpu.sync_copy(dst_hbm, dst_v)
    pltpu.sync_copy(src_hbm, src_v)
    pltpu.make_async_copy(bank_in_hbm.at[pl.ds(lo, _SLAB)],
                          bank_out_hbm.at[pl.ds(lo, _SLAB)], csem).start()
    pltpu.make_async_copy(bank_in_hbm.at[pl.ds(lo, _SLAB)],
                          bank_out_hbm.at[pl.ds(lo, _SLAB)], csem).wait()
    def chunk(c, cnt):
        dstc = dst_v[pl.ds(c * _L, _L)]
        inr = (dstc >= lo) & (dstc < lo + _SLAB)
        mi = jnp.where(inr, 1, 0)
        ccs = mi[0]
        for l in range(1, _L):
            ccs = ccs + mi[l]

        @pl.when(ccs > 0)
        def _issue():
            srcc = src_v[pl.ds(c * _L, _L)]
            for l in range(_L):
                @pl.when(mi[l] > 0)
                def _one():
                    pltpu.make_async_copy(structures_hbm.at[srcc[l]],
                                          bank_out_hbm.at[dstc[l]],
                                          ssem).start()

        return cnt + ccs

    n_issued = lax.fori_loop(0, M // _L, chunk, jnp.int32(0))

    def drain(_, x):
        pltpu.make_async_copy(structures_hbm.at[0], bank_out_hbm.at[0],
                              ssem).wait()
        return x

    lax.fori_loop(0, n_issued, drain, jnp.int32(0))


def _make_sc_kernel():
    mesh = plsc.VectorSubcoreMesh(core_axis_name="c", subcore_axis_name="s")
    return pl.kernel(
        _sc_body,
        out_type=jax.ShapeDtypeStruct((N_E, 128, S7, S7), jnp.float32),
        mesh=mesh,
        scratch_types=[
            pltpu.VMEM((M,), jnp.int32),
            pltpu.VMEM((M,), jnp.int32),
            pltpu.SemaphoreType.DMA,
            pltpu.SemaphoreType.DMA,
        ],
    )


def kernel(z, structures, embedding_weight, structure_bank):
    zf = z.reshape(-1, E_DIM)
    zt = zf.T

    idx_row, _val, loss11 = _tc_argmin(zt, embedding_weight)
    src_col, zq = _tc_dedup_zq(idx_row.reshape(M, 1), idx_row, embedding_weight)

    dst1d = idx_row.reshape(M)
    src1d = src_col.reshape(M)

    new_bank = _make_sc_kernel()(dst1d, src1d, structures, structure_bank)

    loss = loss11[0, 0]
    z_q = zq.reshape(z.shape)
    return (loss, z_q, new_bank)


# speedup vs baseline: 565.0348x; 565.0348x over previous
import functools

import jax
import jax.numpy as jnp
from jax import lax
from jax.experimental import pallas as pl
from jax.experimental.pallas import tpu as pltpu
from jax.experimental.pallas import tpu_sc as plsc

N_E = 8192
E_DIM = 320
S7 = 7
NSP = 49
M = 1024
SROW = 6272
CBLK = 256
NSTEP = N_E // CBLK
BANK_BLK = N_E // NSTEP
BETA = 0.25


def _argmin_body(zt_ref, e_ref, idx_ref, val_ref, loss_ref,
                 zbf_ref, s1_ref, best_val_ref, best_idx_ref):
    j = pl.program_id(0)

    @pl.when(j == 0)
    def _init():
        zt = zt_ref[...]
        zbf_ref[...] = zt.astype(jnp.bfloat16)
        s1_ref[...] = jnp.sum(zt * zt, axis=0, keepdims=True)
        best_val_ref[...] = jnp.full((1, M), jnp.inf, jnp.float32)
        best_idx_ref[...] = jnp.zeros((1, M), jnp.int32)

    e_blk = e_ref[...]
    e_bf = e_blk.astype(jnp.bfloat16)
    mm = lax.dot_general(e_bf, zbf_ref[...], (((1,), (0,)), ((), ())),
                         preferred_element_type=jnp.float32)
    s2 = jnp.sum(e_blk * e_blk, axis=1, keepdims=True)
    d = (s2 + s1_ref[...]) - 2.0 * mm

    blk_min = jnp.min(d, axis=0, keepdims=True)
    row_ids = lax.broadcasted_iota(jnp.int32, (CBLK, M), 0)
    cand = jnp.where(d == blk_min, row_ids, jnp.int32(N_E))
    blk_arg = jnp.min(cand, axis=0, keepdims=True) + j * CBLK

    upd = blk_min < best_val_ref[...]
    best_idx_ref[...] = jnp.where(upd, blk_arg, best_idx_ref[...])
    best_val_ref[...] = jnp.where(upd, blk_min, best_val_ref[...])

    @pl.when(j == NSTEP - 1)
    def _fini():
        idx_ref[...] = best_idx_ref[...]
        val_ref[...] = best_val_ref[...]
        tot = jnp.sum(best_val_ref[...], keepdims=True)
        loss_ref[...] = (1.0 + BETA) * tot.reshape(1, 1) / (M * E_DIM)


def _tc_argmin(zt, emb):
    return pl.pallas_call(
        _argmin_body,
        grid=(NSTEP,),
        in_specs=[
            pl.BlockSpec((E_DIM, M), lambda j: (0, 0)),
            pl.BlockSpec((CBLK, E_DIM), lambda j: (j, 0)),
        ],
        out_specs=[
            pl.BlockSpec((1, M), lambda j: (0, 0)),
            pl.BlockSpec((1, M), lambda j: (0, 0)),
            pl.BlockSpec((1, 1), lambda j: (0, 0)),
        ],
        out_shape=[
            jax.ShapeDtypeStruct((1, M), jnp.int32),
            jax.ShapeDtypeStruct((1, M), jnp.float32),
            jax.ShapeDtypeStruct((1, 1), jnp.float32),
        ],
        scratch_shapes=[
            pltpu.VMEM((E_DIM, M), jnp.bfloat16),
            pltpu.VMEM((1, M), jnp.float32),
            pltpu.VMEM((1, M), jnp.float32),
            pltpu.VMEM((1, M), jnp.int32),
        ],
        compiler_params=pltpu.CompilerParams(
            dimension_semantics=("arbitrary",),
        ),
    )(zt, emb)


ZBLK = 1024
NZSTEP = N_E // ZBLK


def _dedup_zq_body(idx_col_ref, idx_row_ref, e_ref, src_ref, dstp_ref,
                   srcp_ref, zq_ref, acc_ref):
    t = pl.program_id(0)

    @pl.when(t == 0)
    def _dedup():
        col_b = lax.broadcasted_iota(jnp.int32, (M, M), 1)
        eq = idx_col_ref[...] == idx_row_ref[...]
        cand = jnp.where(eq, col_b, jnp.int32(-1))
        src_col = jnp.max(cand, axis=1, keepdims=True)
        src_ref[...] = src_col
        acc_ref[...] = jnp.zeros((M, E_DIM), jnp.float32)
        s_row = lax.broadcasted_iota(jnp.int32, (M, NSP), 1)
        dstp_ref[...] = idx_col_ref[...] + s_row * N_E
        srcp_ref[...] = src_col + s_row * M

    codes = lax.broadcasted_iota(jnp.int32, (1, ZBLK), 1) + t * ZBLK
    oh = (idx_col_ref[...] == codes).astype(jnp.bfloat16)
    e_bf = e_ref[...].astype(jnp.bfloat16)
    acc_ref[...] += lax.dot_general(oh, e_bf, (((1,), (0,)), ((), ())),
                                    preferred_element_type=jnp.float32)

    @pl.when(t == NZSTEP - 1)
    def _fini():
        zq_ref[...] = acc_ref[...]


def _tc_dedup_zq(idx_col, idx_row, emb):
    return pl.pallas_call(
        _dedup_zq_body,
        grid=(NZSTEP,),
        in_specs=[
            pl.BlockSpec((M, 1), lambda t: (0, 0)),
            pl.BlockSpec((1, M), lambda t: (0, 0)),
            pl.BlockSpec((ZBLK, E_DIM), lambda t: (t, 0)),
        ],
        out_specs=[
            pl.BlockSpec((M, 1), lambda t: (0, 0)),
            pl.BlockSpec((M, NSP), lambda t: (0, 0)),
            pl.BlockSpec((M, NSP), lambda t: (0, 0)),
            pl.BlockSpec((M, E_DIM), lambda t: (0, 0)),
        ],
        out_shape=[
            jax.ShapeDtypeStruct((M, 1), jnp.int32),
            jax.ShapeDtypeStruct((M, NSP), jnp.int32),
            jax.ShapeDtypeStruct((M, NSP), jnp.int32),
            jax.ShapeDtypeStruct((M, E_DIM), jnp.float32),
        ],
        scratch_shapes=[pltpu.VMEM((M, E_DIM), jnp.float32)],
        compiler_params=pltpu.CompilerParams(
            dimension_semantics=("arbitrary",),
        ),
    )(idx_col, idx_row, emb)


_NC = 2
_NS = 16
_NW = _NC * _NS
_TPW = M // _NW
_CW = 2 * NSP
_NCHK = 512 // _NW


def _sc_body(dstp_hbm, srcp_hbm, structp_hbm, bankp_ref,
             dst_v, src_v, buf0, buf1, gs0, gs1, ss0, ss1):
    wid = lax.axis_index("s") * _NC + lax.axis_index("c")
    base = wid * _NCHK
    pltpu.sync_copy(dstp_hbm.at[pl.ds(base, _NCHK)], dst_v)
    pltpu.sync_copy(srcp_hbm.at[pl.ds(base, _NCHK)], src_v)
    bufs = (buf0, buf1)
    gsems = (gs0, gs1)
    ssems = (ss0, ss1)
    pltpu.async_copy(structp_hbm.at[src_v.at[0]], buf0, gs0)
    pltpu.async_copy(structp_hbm.at[src_v.at[1]], buf1, gs1)
    for j in range(_NCHK):
        b = j % 2
        pltpu.make_async_copy(structp_hbm.at[src_v.at[j]], bufs[b],
                              gsems[b]).wait()
        pltpu.async_copy(bufs[b], bankp_ref.at[dst_v.at[j]], ssems[b])
        if j + 2 < _NCHK:
            pltpu.make_async_copy(bufs[b], bankp_ref.at[dst_v.at[j]],
                                  ssems[b]).wait()
            pltpu.async_copy(structp_hbm.at[src_v.at[j + 2]], bufs[b],
                             gsems[b])
    for b in range(2):
        pltpu.make_async_copy(bufs[b], bankp_ref.at[dst_v.at[b]],
                              ssems[b]).wait()


def _make_sc_kernel():
    mesh = plsc.VectorSubcoreMesh(core_axis_name="c", subcore_axis_name="s")
    return pl.kernel(
        _sc_body,
        out_type=(),
        mesh=mesh,
        scratch_types=[
            pltpu.VMEM((_NCHK, _CW), jnp.int32),
            pltpu.VMEM((_NCHK, _CW), jnp.int32),
            pltpu.VMEM((_CW, 128), jnp.float32),
            pltpu.VMEM((_CW, 128), jnp.float32),
            pltpu.SemaphoreType.DMA,
            pltpu.SemaphoreType.DMA,
            pltpu.SemaphoreType.DMA,
            pltpu.SemaphoreType.DMA,
        ],
    )


def kernel(z, structures, embedding_weight, structure_bank):
    zf = z.reshape(-1, E_DIM)
    zt = zf.T

    idx_row, _val, loss11 = _tc_argmin(zt, embedding_weight)
    _src_col, dstp, srcp, zq = _tc_dedup_zq(idx_row.reshape(M, 1), idx_row,
                                            embedding_weight)

    dstp2 = dstp.reshape(M // 2, _CW)
    srcp2 = srcp.reshape(M // 2, _CW)
    structp = structures.transpose(2, 3, 0, 1).reshape(NSP * M, 128)
    bankp = structure_bank.transpose(2, 3, 0, 1).reshape(NSP * N_E, 128)

    bank_ref = jax.new_ref(bankp, memory_space=pltpu.HBM)
    _make_sc_kernel()(dstp2, srcp2, structp, bank_ref)
    new_bank = (jax.freeze(bank_ref)
                .reshape(S7, S7, N_E, 128)
                .transpose(2, 3, 0, 1))

    loss = loss11[0, 0]
    z_q = zq.reshape(z.shape)
    return (loss, z_q, new_bank)


# speedup vs baseline: 566.4489x; 1.0025x over previous
import functools

import jax
import jax.numpy as jnp
from jax import lax
from jax.experimental import pallas as pl
from jax.experimental.pallas import tpu as pltpu
from jax.experimental.pallas import tpu_sc as plsc

N_E = 8192
E_DIM = 320
S7 = 7
NSP = 49
M = 1024
SROW = 6272
CBLK = 256
NSTEP = N_E // CBLK
BANK_BLK = N_E // NSTEP
BETA = 0.25


def _argmin_body(zf_ref, e_ref, idx_ref, val_ref, loss_ref,
                 zbf_ref, s1_ref, best_val_ref, best_idx_ref):
    j = pl.program_id(0)

    @pl.when(j == 0)
    def _init():
        zf = zf_ref[...]
        zbf_ref[...] = zf.astype(jnp.bfloat16)
        z2 = (zf * zf).astype(jnp.bfloat16)
        ones = jnp.ones((1, E_DIM), jnp.bfloat16)
        s1_ref[...] = lax.dot_general(ones, z2, (((1,), (1,)), ((), ())),
                                      preferred_element_type=jnp.float32)
        best_val_ref[...] = jnp.full((1, M), jnp.inf, jnp.float32)
        best_idx_ref[...] = jnp.zeros((1, M), jnp.int32)

    e_blk = e_ref[...]
    e_bf = e_blk.astype(jnp.bfloat16)
    mm = lax.dot_general(e_bf, zbf_ref[...], (((1,), (1,)), ((), ())),
                         preferred_element_type=jnp.float32)
    s2 = jnp.sum(e_blk * e_blk, axis=1, keepdims=True)
    d = (s2 + s1_ref[...]) - 2.0 * mm

    blk_min = jnp.min(d, axis=0, keepdims=True)
    row_ids = lax.broadcasted_iota(jnp.int32, (CBLK, M), 0)
    cand = jnp.where(d == blk_min, row_ids, jnp.int32(N_E))
    blk_arg = jnp.min(cand, axis=0, keepdims=True) + j * CBLK

    upd = blk_min < best_val_ref[...]
    best_idx_ref[...] = jnp.where(upd, blk_arg, best_idx_ref[...])
    best_val_ref[...] = jnp.where(upd, blk_min, best_val_ref[...])

    @pl.when(j == NSTEP - 1)
    def _fini():
        idx_ref[...] = best_idx_ref[...]
        val_ref[...] = best_val_ref[...]
        tot = jnp.sum(best_val_ref[...], keepdims=True)
        loss_ref[...] = (1.0 + BETA) * tot.reshape(1, 1) / (M * E_DIM)


def _tc_argmin(zf, emb):
    return pl.pallas_call(
        _argmin_body,
        grid=(NSTEP,),
        in_specs=[
            pl.BlockSpec((M, E_DIM), lambda j: (0, 0)),
            pl.BlockSpec((CBLK, E_DIM), lambda j: (j, 0)),
        ],
        out_specs=[
            pl.BlockSpec((1, M), lambda j: (0, 0)),
            pl.BlockSpec((1, M), lambda j: (0, 0)),
            pl.BlockSpec((1, 1), lambda j: (0, 0)),
        ],
        out_shape=[
            jax.ShapeDtypeStruct((1, M), jnp.int32),
            jax.ShapeDtypeStruct((1, M), jnp.float32),
            jax.ShapeDtypeStruct((1, 1), jnp.float32),
        ],
        scratch_shapes=[
            pltpu.VMEM((M, E_DIM), jnp.bfloat16),
            pltpu.VMEM((1, M), jnp.float32),
            pltpu.VMEM((1, M), jnp.float32),
            pltpu.VMEM((1, M), jnp.int32),
        ],
        compiler_params=pltpu.CompilerParams(
            dimension_semantics=("arbitrary",),
        ),
    )(zf, emb)


ZBLK = 1024
NZSTEP = N_E // ZBLK


def _dedup_zq_body(idx_col_ref, idx_row_ref, e_ref, src_ref, dstp_ref,
                   srcp_ref, zq_ref, acc_ref):
    t = pl.program_id(0)

    @pl.when(t == 0)
    def _dedup():
        col_b = lax.broadcasted_iota(jnp.int32, (M, M), 1)
        eq = idx_col_ref[...] == idx_row_ref[...]
        cand = jnp.where(eq, col_b, jnp.int32(-1))
        src_col = jnp.max(cand, axis=1, keepdims=True)
        src_ref[...] = src_col
        acc_ref[...] = jnp.zeros((M, E_DIM), jnp.float32)
        s_row = lax.broadcasted_iota(jnp.int32, (M, NSP), 1)
        dstp_ref[...] = idx_col_ref[...] + s_row * N_E
        srcp_ref[...] = src_col + s_row * M

    codes = lax.broadcasted_iota(jnp.int32, (1, ZBLK), 1) + t * ZBLK
    oh = (idx_col_ref[...] == codes).astype(jnp.bfloat16)
    e_bf = e_ref[...].astype(jnp.bfloat16)
    acc_ref[...] += lax.dot_general(oh, e_bf, (((1,), (0,)), ((), ())),
                                    preferred_element_type=jnp.float32)

    @pl.when(t == NZSTEP - 1)
    def _fini():
        zq_ref[...] = acc_ref[...]


def _tc_dedup_zq(idx_col, idx_row, emb):
    return pl.pallas_call(
        _dedup_zq_body,
        grid=(NZSTEP,),
        in_specs=[
            pl.BlockSpec((M, 1), lambda t: (0, 0)),
            pl.BlockSpec((1, M), lambda t: (0, 0)),
            pl.BlockSpec((ZBLK, E_DIM), lambda t: (t, 0)),
        ],
        out_specs=[
            pl.BlockSpec((M, 1), lambda t: (0, 0)),
            pl.BlockSpec((M, NSP), lambda t: (0, 0)),
            pl.BlockSpec((M, NSP), lambda t: (0, 0)),
            pl.BlockSpec((M, E_DIM), lambda t: (0, 0)),
        ],
        out_shape=[
            jax.ShapeDtypeStruct((M, 1), jnp.int32),
            jax.ShapeDtypeStruct((M, NSP), jnp.int32),
            jax.ShapeDtypeStruct((M, NSP), jnp.int32),
            jax.ShapeDtypeStruct((M, E_DIM), jnp.float32),
        ],
        scratch_shapes=[pltpu.VMEM((M, E_DIM), jnp.float32)],
        compiler_params=pltpu.CompilerParams(
            dimension_semantics=("arbitrary",),
        ),
    )(idx_col, idx_row, emb)


_NC = 2
_NS = 16
_NW = _NC * _NS
_TPW = M // _NW
_CW = 2 * NSP
_NCHK = 512 // _NW


def _sc_body(dstp_hbm, srcp_hbm, structp_hbm, bankp_ref,
             dst_v, src_v, buf0, buf1, gs0, gs1, ss0, ss1):
    wid = lax.axis_index("s") * _NC + lax.axis_index("c")
    base = wid * _NCHK
    pltpu.sync_copy(dstp_hbm.at[pl.ds(base, _NCHK)], dst_v)
    pltpu.sync_copy(srcp_hbm.at[pl.ds(base, _NCHK)], src_v)
    bufs = (buf0, buf1)
    gsems = (gs0, gs1)
    ssems = (ss0, ss1)
    pltpu.async_copy(structp_hbm.at[src_v.at[0]], buf0, gs0)
    pltpu.async_copy(structp_hbm.at[src_v.at[1]], buf1, gs1)
    for j in range(_NCHK):
        b = j % 2
        pltpu.make_async_copy(structp_hbm.at[src_v.at[j]], bufs[b],
                              gsems[b]).wait()
        pltpu.async_copy(bufs[b], bankp_ref.at[dst_v.at[j]], ssems[b])
        if j + 2 < _NCHK:
            pltpu.make_async_copy(bufs[b], bankp_ref.at[dst_v.at[j]],
                                  ssems[b]).wait()
            pltpu.async_copy(structp_hbm.at[src_v.at[j + 2]], bufs[b],
                             gsems[b])
    for b in range(2):
        pltpu.make_async_copy(bufs[b], bankp_ref.at[dst_v.at[b]],
                              ssems[b]).wait()


def _make_sc_kernel():
    mesh = plsc.VectorSubcoreMesh(core_axis_name="c", subcore_axis_name="s")
    return pl.kernel(
        _sc_body,
        out_type=(),
        mesh=mesh,
        scratch_types=[
            pltpu.VMEM((_NCHK, _CW), jnp.int32),
            pltpu.VMEM((_NCHK, _CW), jnp.int32),
            pltpu.VMEM((_CW, 128), jnp.float32),
            pltpu.VMEM((_CW, 128), jnp.float32),
            pltpu.SemaphoreType.DMA,
            pltpu.SemaphoreType.DMA,
            pltpu.SemaphoreType.DMA,
            pltpu.SemaphoreType.DMA,
        ],
    )


def kernel(z, structures, embedding_weight, structure_bank):
    zf = z.reshape(-1, E_DIM)

    idx_row, _val, loss11 = _tc_argmin(zf, embedding_weight)
    _src_col, dstp, srcp, zq = _tc_dedup_zq(idx_row.reshape(M, 1), idx_row,
                                            embedding_weight)

    dstp2 = dstp.reshape(M // 2, _CW)
    srcp2 = srcp.reshape(M // 2, _CW)
    structp = structures.transpose(2, 3, 0, 1).reshape(NSP * M, 128)
    bankp = structure_bank.transpose(2, 3, 0, 1).reshape(NSP * N_E, 128)

    bank_ref = jax.new_ref(bankp, memory_space=pltpu.HBM)
    _make_sc_kernel()(dstp2, srcp2, structp, bank_ref)
    new_bank = (jax.freeze(bank_ref)
                .reshape(S7, S7, N_E, 128)
                .transpose(2, 3, 0, 1))

    loss = loss11[0, 0]
    z_q = zq.reshape(z.shape)
    return (loss, z_q, new_bank)
